# baseline (speedup 1.0000x reference)
import functools

import jax
import jax.numpy as jnp
from jax.experimental import pallas as pl

_BLK = 262144

_KS0 = 0
_KS1 = 42
_KS2 = _KS0 ^ _KS1 ^ 0x1BD11BDA

_ROTS = ((13, 15, 26, 6), (17, 29, 16, 24))


def _rotl(x, r):
    return (x << jnp.uint32(r)) | (x >> jnp.uint32(32 - r))


def _dropout_blk_kernel(iota_ref, v_ref, o_ref):
    pid = pl.program_id(0).astype(jnp.uint32)
    x1 = iota_ref[...] + (pid * jnp.uint32(_BLK) + jnp.uint32(_KS1))
    x0 = jnp.zeros_like(x1)
    ks = (jnp.uint32(_KS0), jnp.uint32(_KS1), jnp.uint32(_KS2))
    for i in range(5):
        for r in _ROTS[i % 2]:
            x0 = x0 + x1
            x1 = _rotl(x1, r)
            x1 = x0 ^ x1
        x0 = x0 + ks[(i + 1) % 3]
        x1 = x1 + ks[(i + 2) % 3] + jnp.uint32(i + 1)
    bits = x0 ^ x1
    keep = bits < jnp.uint32(0x80000000)
    v = v_ref[...]
    o_ref[...] = jnp.where(keep, v * jnp.float32(2.0), jnp.float32(0.0))


@functools.partial(jax.jit, static_argnames=())
def _sparse_dropout(values):
    n = values.shape[0]
    grid = pl.cdiv(n, _BLK)
    iota = jnp.arange(_BLK, dtype=jnp.uint32)
    return pl.pallas_call(
        _dropout_blk_kernel,
        grid=(grid,),
        in_specs=[
            pl.BlockSpec((_BLK,), lambda i: (0,)),
            pl.BlockSpec((_BLK,), lambda i: (i,)),
        ],
        out_specs=pl.BlockSpec((_BLK,), lambda i: (i,)),
        out_shape=jax.ShapeDtypeStruct((n,), jnp.float32),
    )(iota, values)


def kernel(indices, values):
    del indices
    return _sparse_dropout(values)


# speedup vs baseline: 1.0195x; 1.0195x over previous
import functools

import jax
import jax.numpy as jnp
from jax.experimental import pallas as pl

_BLK = 65536

_KS0 = 0
_KS1 = 42
_KS2 = _KS0 ^ _KS1 ^ 0x1BD11BDA

_ROTS = ((13, 15, 26, 6), (17, 29, 16, 24))


def _rotl(x, r):
    return (x << jnp.uint32(r)) | (x >> jnp.uint32(32 - r))


def _dropout_blk_kernel(iota_ref, v_ref, o_ref):
    pid = pl.program_id(0).astype(jnp.uint32)
    x1 = iota_ref[...] + (pid * jnp.uint32(_BLK) + jnp.uint32(_KS1))
    x0 = jnp.zeros_like(x1)
    ks = (jnp.uint32(_KS0), jnp.uint32(_KS1), jnp.uint32(_KS2))
    for i in range(5):
        for r in _ROTS[i % 2]:
            x0 = x0 + x1
            x1 = _rotl(x1, r)
            x1 = x0 ^ x1
        x0 = x0 + ks[(i + 1) % 3]
        x1 = x1 + ks[(i + 2) % 3] + jnp.uint32(i + 1)
    bits = x0 ^ x1
    keep = bits < jnp.uint32(0x80000000)
    v = v_ref[...]
    o_ref[...] = jnp.where(keep, v * jnp.float32(2.0), jnp.float32(0.0))


@functools.partial(jax.jit, static_argnames=())
def _sparse_dropout(values):
    n = values.shape[0]
    grid = pl.cdiv(n, _BLK)
    iota = jnp.arange(_BLK, dtype=jnp.uint32)
    return pl.pallas_call(
        _dropout_blk_kernel,
        grid=(grid,),
        in_specs=[
            pl.BlockSpec((_BLK,), lambda i: (0,)),
            pl.BlockSpec((_BLK,), lambda i: (i,)),
        ],
        out_specs=pl.BlockSpec((_BLK,), lambda i: (i,)),
        out_shape=jax.ShapeDtypeStruct((n,), jnp.float32),
    )(iota, values)


def kernel(indices, values):
    del indices
    return _sparse_dropout(values)
